# baseline (speedup 1.0000x reference)
import functools

import jax
import jax.numpy as jnp
from jax import lax
from jax.experimental import pallas as pl
from jax.experimental.pallas import tpu as pltpu
from jax.experimental.pallas import tpu_sc as plsc

N = 10000
E = 160000
D = 256
DQ = 64
NCLASS = 64
NC = 2
NS = 16
NPAD = 10240
RPS = NPAD // NS
KC = 80
NCHUNK = 130
EPAD = NS * NCHUNK * KC
EPS = EPAD // NS
NBUF = 10
NGRP = NCHUNK // NBUF
GAHEAD = 8
BLK = 2048
GRID = NPAD // BLK


def _sc_mesh():
    return plsc.VectorSubcoreMesh(core_axis_name="c", subcore_axis_name="s")


@functools.partial(
    pl.kernel,
    out_type=(jax.ShapeDtypeStruct((NPAD,), jnp.float32),
              jax.ShapeDtypeStruct((NPAD,), jnp.float32)),
    mesh=_sc_mesh(),
    scratch_types=(
        pltpu.VMEM((NCHUNK, KC), jnp.int32),
        pltpu.VMEM((KC,), jnp.float32),
        pltpu.VMEM((RPS,), jnp.float32),
        pltpu.VMEM_SHARED((NPAD,), jnp.float32),
        pltpu.SemaphoreType.DMA,
    ),
    compiler_params=pltpu.CompilerParams(use_tc_tiling_on_sc=False),
)
def _sc_degrees(src_hbm, dst_hbm, ones_hbm, zrow_hbm,
                degout_hbm, degin_hbm,
                idxbuf, ones_v, bounce, degsp, sem):
    cid = lax.axis_index("c")
    sid = lax.axis_index("s")
    pltpu.sync_copy(ones_hbm, ones_v)
    pltpu.sync_copy(zrow_hbm, bounce)
    pltpu.sync_copy(bounce, degsp.at[pl.ds(sid * RPS, RPS)])
    plsc.subcore_barrier()

    def run(idx_hbm, out_hbm):
        pltpu.sync_copy(idx_hbm.at[sid], idxbuf)

        def body(i, carry):
            pltpu.async_copy(ones_v, degsp.at[idxbuf.at[i]], sem, add=True)
            return carry

        lax.fori_loop(0, NCHUNK, body, 0)

        def drain(i, carry):
            pltpu.make_async_copy(ones_v, degsp.at[idxbuf.at[i]], sem).wait()
            return carry

        lax.fori_loop(0, NCHUNK, drain, 0)
        plsc.subcore_barrier()
        pltpu.sync_copy(degsp.at[pl.ds(sid * RPS, RPS)],
                        out_hbm.at[pl.ds(sid * RPS, RPS)])

    @pl.when(cid == 0)
    def _():
        run(src_hbm, degout_hbm)

    @pl.when(cid == 1)
    def _():
        run(dst_hbm, degin_hbm)


@functools.partial(
    pl.kernel,
    out_type=jax.ShapeDtypeStruct((NPAD, D), jnp.float32),
    mesh=_sc_mesh(),
    scratch_types=(
        pltpu.VMEM((NCHUNK, KC), jnp.int32),
        pltpu.VMEM((NCHUNK, KC), jnp.int32),
        [pltpu.VMEM((KC, DQ), jnp.float32)] * NBUF,
        pltpu.VMEM((RPS // 4, DQ), jnp.float32),
        pltpu.VMEM_SHARED((NPAD, DQ), jnp.float32),
        [pltpu.SemaphoreType.DMA] * NBUF,
        [pltpu.SemaphoreType.DMA] * NBUF,
    ),
    compiler_params=pltpu.CompilerParams(use_tc_tiling_on_sc=False),
)
def _sc_gather_scatter(x0_hbm, x1_hbm, x2_hbm, x3_hbm, src_hbm, dst_hbm,
                       zpad_hbm, agg_hbm,
                       srcbuf, dstbuf, rows, bounce, aggsp, gsem, ssem):
    cid = lax.axis_index("c")
    sid = lax.axis_index("s")
    pltpu.sync_copy(src_hbm.at[sid], srcbuf)
    pltpu.sync_copy(dst_hbm.at[sid], dstbuf)
    pltpu.sync_copy(zpad_hbm, bounce)

    def run(tab_hbm, qoff):
        for z in range(4):
            pltpu.sync_copy(
                bounce, aggsp.at[pl.ds(sid * RPS + z * (RPS // 4), RPS // 4)])
        plsc.subcore_barrier()
        for b in range(GAHEAD):
            pltpu.async_copy(tab_hbm.at[srcbuf.at[b]], rows[b], gsem[b])

        def body(g, carry):
            for b in range(NBUF):
                i = g * NBUF + b
                pltpu.make_async_copy(tab_hbm.at[srcbuf.at[i]], rows[b],
                                      gsem[b]).wait()
                j = i + GAHEAD
                bj = (b + GAHEAD) % NBUF

                @pl.when(j < NCHUNK)
                def _():
                    @pl.when(i >= NBUF - GAHEAD)
                    def _():
                        pltpu.make_async_copy(
                            rows[bj], aggsp.at[dstbuf.at[j - NBUF]],
                            ssem[bj]).wait()

                    pltpu.async_copy(tab_hbm.at[srcbuf.at[j]], rows[bj],
                                     gsem[bj])

                pltpu.async_copy(rows[b], aggsp.at[dstbuf.at[i]], ssem[b],
                                 add=True)
            return carry

        lax.fori_loop(0, NGRP, body, 0)
        for b in range(NBUF):
            i = NCHUNK - NBUF + b
            pltpu.make_async_copy(rows[b], aggsp.at[dstbuf.at[i]],
                                  ssem[b]).wait()
        plsc.subcore_barrier()
        pltpu.sync_copy(aggsp.at[pl.ds(sid * RPS, RPS)],
                        agg_hbm.at[pl.ds(sid * RPS, RPS), pl.ds(qoff, DQ)])

    @pl.when(cid == 0)
    def _():
        run(x0_hbm, 0)
        run(x1_hbm, DQ)

    @pl.when(cid == 1)
    def _():
        run(x2_hbm, 2 * DQ)
        run(x3_hbm, 3 * DQ)


def _split4(x_ref, out_refs):
    for q, ref in enumerate(out_refs):
        ref[...] = x_ref[:, q * DQ:(q + 1) * DQ]


def _tc_in_body(h_ref, w_ref, b_ref, dego_ref, *out_refs):
    x = jnp.dot(h_ref[...], w_ref[...], preferred_element_type=jnp.float32)
    x = jnp.maximum(x + b_ref[...], 0.0)
    ns = lax.rsqrt(jnp.maximum(dego_ref[...], 1.0))
    _split4(x * ns, out_refs)


def _tc_mid_body(agg_ref, w_ref, b_ref, degi_ref,
                 dego_ref, *out_refs):
    nd = lax.rsqrt(jnp.maximum(degi_ref[...], 1.0))
    agg = agg_ref[...] * nd
    x = jnp.dot(agg, w_ref[...], preferred_element_type=jnp.float32)
    x = jnp.maximum(x + b_ref[...], 0.0)
    ns = lax.rsqrt(jnp.maximum(dego_ref[...], 1.0))
    _split4(x * ns, out_refs)


def _tc_out_body(agg_ref, w_ref, b_ref, wo_ref,
                 bo_ref, degi_ref, out_ref):
    nd = lax.rsqrt(jnp.maximum(degi_ref[...], 1.0))
    agg = agg_ref[...] * nd
    x = jnp.dot(agg, w_ref[...], preferred_element_type=jnp.float32)
    x = jnp.maximum(x + b_ref[...], 0.0)
    z = jnp.dot(x, wo_ref[...], preferred_element_type=jnp.float32)
    z = z + bo_ref[...]
    m = jnp.max(z, axis=1, keepdims=True)
    lse = m + jnp.log(jnp.sum(jnp.exp(z - m), axis=1, keepdims=True))
    out_ref[...] = z - lse


_q_spec = pl.BlockSpec((BLK, DQ), lambda i: (i, 0))
_deg_spec = pl.BlockSpec((BLK, 1), lambda i: (i, 0))
_w_spec = pl.BlockSpec((D, D), lambda i: (0, 0))
_b_spec = pl.BlockSpec((1, D), lambda i: (0, 0))
_q_shape = jax.ShapeDtypeStruct((NPAD, DQ), jnp.float32)

_tc_in = pl.pallas_call(
    _tc_in_body,
    grid=(GRID,),
    in_specs=[pl.BlockSpec((BLK, D), lambda i: (i, 0)), _w_spec, _b_spec,
              _deg_spec],
    out_specs=[_q_spec] * 4,
    out_shape=[_q_shape] * 4,
)

_tc_mid = pl.pallas_call(
    _tc_mid_body,
    grid=(GRID,),
    in_specs=[pl.BlockSpec((BLK, D), lambda i: (i, 0)), _w_spec, _b_spec,
              _deg_spec, _deg_spec],
    out_specs=[_q_spec] * 4,
    out_shape=[_q_shape] * 4,
)

_tc_out = pl.pallas_call(
    _tc_out_body,
    grid=(GRID,),
    in_specs=[pl.BlockSpec((BLK, D), lambda i: (i, 0)),
        _w_spec, _b_spec,
        pl.BlockSpec((D, NCLASS), lambda i: (0, 0)),
        pl.BlockSpec((1, NCLASS), lambda i: (0, 0)),
        _deg_spec],
    out_specs=pl.BlockSpec((BLK, NCLASS), lambda i: (i, 0)),
    out_shape=jax.ShapeDtypeStruct((NPAD, NCLASS), jnp.float32),
)


def kernel(h, edge_index, W_in, b_in, W0, b0, W1, b1, W2, b2, W_out, b_out):
    epad = jnp.full((EPAD - E,), N, jnp.int32)
    src = jnp.concatenate([edge_index[0], epad]).reshape(NS, NCHUNK, KC)
    dst = jnp.concatenate([edge_index[1], epad]).reshape(NS, NCHUNK, KC)
    ones_kc = jnp.ones((KC,), jnp.float32)
    zrow = jnp.zeros((RPS,), jnp.float32)
    zpad = jnp.zeros((RPS // 4, DQ), jnp.float32)
    h_pad = jnp.pad(h, ((0, NPAD - N), (0, 0)))

    dego, degi = _sc_degrees(src, dst, ones_kc, zrow)
    dego = dego.reshape(NPAD, 1)
    degi = degi.reshape(NPAD, 1)

    xq = _tc_in(h_pad, W_in, b_in.reshape(1, D), dego)
    for W, b in ((W0, b0), (W1, b1)):
        agg = _sc_gather_scatter(*xq, src, dst, zpad)
        xq = _tc_mid(agg, W, b.reshape(1, D), degi, dego)
    agg = _sc_gather_scatter(*xq, src, dst, zpad)
    out = _tc_out(agg, W2, b2.reshape(1, D), W_out,
                  b_out.reshape(1, NCLASS), degi)
    return out[:N]


# speedup vs baseline: 2.9753x; 2.9753x over previous
import functools

import jax
import jax.numpy as jnp
from jax import lax
from jax.experimental import pallas as pl
from jax.experimental.pallas import tpu as pltpu
from jax.experimental.pallas import tpu_sc as plsc

N = 10000
E = 160000
D = 256
DQ = 64
NCLASS = 64
NC = 2
NS = 16
NPAD = 10240
RPS = NPAD // NS
EPS = E // NS
KC = 80
NCHUNK = EPS // KC
NBUF = 5
NGRP = NCHUNK // NBUF
GAHEAD = 4
BLK = 2048
GRID = NPAD // BLK


def _sc_mesh():
    return plsc.VectorSubcoreMesh(core_axis_name="c", subcore_axis_name="s")


@functools.partial(
    pl.kernel,
    out_type=(jax.ShapeDtypeStruct((NPAD,), jnp.float32),
              jax.ShapeDtypeStruct((NPAD,), jnp.float32)),
    mesh=_sc_mesh(),
    scratch_types=(
        pltpu.VMEM((NCHUNK, KC), jnp.int32),
        pltpu.VMEM((KC,), jnp.float32),
        pltpu.VMEM((RPS,), jnp.float32),
        pltpu.VMEM_SHARED((NPAD,), jnp.float32),
        pltpu.SemaphoreType.DMA,
    ),
    compiler_params=pltpu.CompilerParams(use_tc_tiling_on_sc=False),
)
def _sc_degrees(src_hbm, dst_hbm, ones_hbm, zrow_hbm,
                degout_hbm, degin_hbm,
                idxbuf, ones_v, bounce, degsp, sem):
    cid = lax.axis_index("c")
    sid = lax.axis_index("s")
    pltpu.sync_copy(ones_hbm, ones_v)
    pltpu.sync_copy(zrow_hbm, bounce)
    pltpu.sync_copy(bounce, degsp.at[pl.ds(sid * RPS, RPS)])
    plsc.subcore_barrier()

    def run(idx_hbm, out_hbm):
        pltpu.sync_copy(idx_hbm.at[sid], idxbuf)

        def body(i, carry):
            pltpu.async_copy(ones_v, degsp.at[idxbuf.at[i]], sem, add=True)
            return carry

        lax.fori_loop(0, NCHUNK, body, 0)

        def drain(i, carry):
            pltpu.make_async_copy(ones_v, degsp.at[idxbuf.at[i]], sem).wait()
            return carry

        lax.fori_loop(0, NCHUNK, drain, 0)
        plsc.subcore_barrier()
        pltpu.sync_copy(degsp.at[pl.ds(sid * RPS, RPS)],
                        out_hbm.at[pl.ds(sid * RPS, RPS)])

    @pl.when(cid == 0)
    def _():
        run(src_hbm, degout_hbm)

    @pl.when(cid == 1)
    def _():
        run(dst_hbm, degin_hbm)


@functools.partial(
    pl.kernel,
    out_type=jax.ShapeDtypeStruct((NPAD, D), jnp.float32),
    mesh=_sc_mesh(),
    scratch_types=(
        pltpu.VMEM((NCHUNK, KC), jnp.int32),
        pltpu.VMEM((NCHUNK, KC), jnp.int32),
        [pltpu.VMEM((KC, DQ), jnp.float32)] * NBUF,
        pltpu.VMEM((RPS, DQ), jnp.float32),
        pltpu.VMEM_SHARED((NPAD, DQ), jnp.float32),
        [pltpu.SemaphoreType.DMA] * NBUF,
        [pltpu.SemaphoreType.DMA] * NBUF,
    ),
    compiler_params=pltpu.CompilerParams(use_tc_tiling_on_sc=False),
)
def _sc_gather_scatter(x0_hbm, x1_hbm, x2_hbm, x3_hbm, src_hbm, dst_hbm,
                       zpad_hbm, agg_hbm,
                       srcbuf, dstbuf, rows, bounce, aggsp, gsem, ssem):
    cid = lax.axis_index("c")
    sid = lax.axis_index("s")
    pltpu.sync_copy(src_hbm.at[sid], srcbuf)
    pltpu.sync_copy(dst_hbm.at[sid], dstbuf)
    pltpu.sync_copy(zpad_hbm, bounce)

    def run(tab_hbm, qoff):
        pltpu.sync_copy(bounce, aggsp.at[pl.ds(sid * RPS, RPS)])
        plsc.subcore_barrier()
        for b in range(GAHEAD):
            pltpu.async_copy(tab_hbm.at[srcbuf.at[b]], rows[b], gsem[b])

        def body(g, carry):
            for b in range(NBUF):
                i = g * NBUF + b
                pltpu.make_async_copy(tab_hbm.at[srcbuf.at[i]], rows[b],
                                      gsem[b]).wait()
                j = i + GAHEAD
                bj = (b + GAHEAD) % NBUF

                @pl.when(j < NCHUNK)
                def _():
                    @pl.when(i >= NBUF - GAHEAD)
                    def _():
                        pltpu.make_async_copy(
                            rows[bj], aggsp.at[dstbuf.at[j - NBUF]],
                            ssem[bj]).wait()

                    pltpu.async_copy(tab_hbm.at[srcbuf.at[j]], rows[bj],
                                     gsem[bj])

                pltpu.async_copy(rows[b], aggsp.at[dstbuf.at[i]], ssem[b],
                                 add=True)
            return carry

        lax.fori_loop(0, NGRP, body, 0)
        for b in range(NBUF):
            i = NCHUNK - NBUF + b
            pltpu.make_async_copy(rows[b], aggsp.at[dstbuf.at[i]],
                                  ssem[b]).wait()
        plsc.subcore_barrier()
        pltpu.sync_copy(aggsp.at[pl.ds(sid * RPS, RPS)],
                        agg_hbm.at[pl.ds(sid * RPS, RPS), pl.ds(qoff, DQ)])

    @pl.when(cid == 0)
    def _():
        run(x0_hbm, 0)
        run(x1_hbm, DQ)

    @pl.when(cid == 1)
    def _():
        run(x2_hbm, 2 * DQ)
        run(x3_hbm, 3 * DQ)


def _split4(x_ref, out_refs):
    for q, ref in enumerate(out_refs):
        ref[...] = x_ref[:, q * DQ:(q + 1) * DQ]


def _tc_in_body(h_ref, w_ref, b_ref, dego_ref, *out_refs):
    x = jnp.dot(h_ref[...], w_ref[...], preferred_element_type=jnp.float32)
    x = jnp.maximum(x + b_ref[...], 0.0)
    ns = lax.rsqrt(jnp.maximum(dego_ref[...], 1.0))
    _split4(x * ns, out_refs)


def _tc_mid_body(agg_ref, w_ref, b_ref, degi_ref,
                 dego_ref, *out_refs):
    nd = lax.rsqrt(jnp.maximum(degi_ref[...], 1.0))
    agg = agg_ref[...] * nd
    x = jnp.dot(agg, w_ref[...], preferred_element_type=jnp.float32)
    x = jnp.maximum(x + b_ref[...], 0.0)
    ns = lax.rsqrt(jnp.maximum(dego_ref[...], 1.0))
    _split4(x * ns, out_refs)


def _tc_out_body(agg_ref, w_ref, b_ref, wo_ref,
                 bo_ref, degi_ref, out_ref):
    nd = lax.rsqrt(jnp.maximum(degi_ref[...], 1.0))
    agg = agg_ref[...] * nd
    x = jnp.dot(agg, w_ref[...], preferred_element_type=jnp.float32)
    x = jnp.maximum(x + b_ref[...], 0.0)
    z = jnp.dot(x, wo_ref[...], preferred_element_type=jnp.float32)
    z = z + bo_ref[...]
    m = jnp.max(z, axis=1, keepdims=True)
    lse = m + jnp.log(jnp.sum(jnp.exp(z - m), axis=1, keepdims=True))
    out_ref[...] = z - lse


_q_spec = pl.BlockSpec((BLK, DQ), lambda i: (i, 0))
_deg_spec = pl.BlockSpec((BLK, 1), lambda i: (i, 0))
_w_spec = pl.BlockSpec((D, D), lambda i: (0, 0))
_b_spec = pl.BlockSpec((1, D), lambda i: (0, 0))
_q_shape = jax.ShapeDtypeStruct((NPAD, DQ), jnp.float32)

_tc_in = pl.pallas_call(
    _tc_in_body,
    grid=(GRID,),
    in_specs=[pl.BlockSpec((BLK, D), lambda i: (i, 0)), _w_spec, _b_spec,
              _deg_spec],
    out_specs=[_q_spec] * 4,
    out_shape=[_q_shape] * 4,
)

_tc_mid = pl.pallas_call(
    _tc_mid_body,
    grid=(GRID,),
    in_specs=[pl.BlockSpec((BLK, D), lambda i: (i, 0)), _w_spec, _b_spec,
              _deg_spec, _deg_spec],
    out_specs=[_q_spec] * 4,
    out_shape=[_q_shape] * 4,
)

_tc_out = pl.pallas_call(
    _tc_out_body,
    grid=(GRID,),
    in_specs=[pl.BlockSpec((BLK, D), lambda i: (i, 0)),
        _w_spec, _b_spec,
        pl.BlockSpec((D, NCLASS), lambda i: (0, 0)),
        pl.BlockSpec((1, NCLASS), lambda i: (0, 0)),
        _deg_spec],
    out_specs=pl.BlockSpec((BLK, NCLASS), lambda i: (i, 0)),
    out_shape=jax.ShapeDtypeStruct((NPAD, NCLASS), jnp.float32),
)


def kernel(h, edge_index, W_in, b_in, W0, b0, W1, b1, W2, b2, W_out, b_out):
    src = edge_index[0].reshape(NS, NCHUNK, KC)
    dst = edge_index[1].reshape(NS, NCHUNK, KC)
    ones_kc = jnp.ones((KC,), jnp.float32)
    zrow = jnp.zeros((RPS,), jnp.float32)
    zpad = jnp.zeros((RPS, DQ), jnp.float32)
    h_pad = jnp.pad(h, ((0, NPAD - N), (0, 0)))

    dego, degi = _sc_degrees(src, dst, ones_kc, zrow)
    dego = dego.reshape(NPAD, 1)
    degi = degi.reshape(NPAD, 1)

    xq = _tc_in(h_pad, W_in, b_in.reshape(1, D), dego)
    for W, b in ((W0, b0), (W1, b1)):
        agg = _sc_gather_scatter(*xq, src, dst, zpad)
        xq = _tc_mid(agg, W, b.reshape(1, D), degi, dego)
    agg = _sc_gather_scatter(*xq, src, dst, zpad)
    out = _tc_out(agg, W2, b2.reshape(1, D), W_out,
                  b_out.reshape(1, NCLASS), degi)
    return out[:N]
